# Initial kernel scaffold; baseline (speedup 1.0000x reference)
import jax
import jax.numpy as jnp
from jax.experimental import pallas as pl


def kernel(x):
    raise NotImplementedError("write your pallas kernel here")


# speedup vs baseline: 65.2882x; 65.2882x over previous
import jax
import jax.numpy as jnp
from jax.experimental import pallas as pl
from jax.experimental.pallas import tpu as pltpu

_EPS = 1e-6
_S = 0.025
_ALPHA = 0.98
_DELTA = 2.0
_R = 0.5

_TC = 512


def _pcen_kernel(x_ref, o_ref, m_ref):
    @pl.when(pl.program_id(1) == 0)
    def _init():
        m_ref[...] = x_ref[:, 0, :]

    neg_droot = -(_DELTA ** _R)

    def body(t, m):
        xv = x_ref[:, t, :]
        m = (1.0 - _S) * m + _S * xv
        denom = jnp.exp(_ALPHA * jnp.log(m + _EPS))
        o_ref[:, t, :] = jnp.sqrt(xv / denom + _DELTA) + neg_droot
        return m

    m_ref[...] = jax.lax.fori_loop(0, _TC, body, m_ref[...], unroll=8)


@jax.jit
def kernel(x):
    B, T, F = x.shape
    n_cores = 2
    bs = B // n_cores
    grid = (n_cores, T // _TC)
    return pl.pallas_call(
        _pcen_kernel,
        grid=grid,
        in_specs=[pl.BlockSpec((bs, _TC, F), lambda b, t: (b, t, 0))],
        out_specs=pl.BlockSpec((bs, _TC, F), lambda b, t: (b, t, 0)),
        out_shape=jax.ShapeDtypeStruct((B, T, F), x.dtype),
        scratch_shapes=[pltpu.VMEM((bs, F), jnp.float32)],
        compiler_params=pltpu.CompilerParams(
            dimension_semantics=("parallel", "arbitrary"),
        ),
    )(x)


# speedup vs baseline: 167.3550x; 2.5633x over previous
import math

import jax
import jax.numpy as jnp
from jax.experimental import pallas as pl
from jax.experimental.pallas import tpu as pltpu

_EPS = 1e-6
_S = 0.025
_ALPHA = 0.98
_DELTA = 2.0
_R = 0.5

_TILE = 128
_TC = 1024


def _pcen_kernel(x_ref, o_ref, carry_ref):
    f32 = jnp.float32
    log1ms = math.log(1.0 - _S)

    row = jax.lax.broadcasted_iota(jnp.int32, (_TILE, _TILE), 0)
    col = jax.lax.broadcasted_iota(jnp.int32, (_TILE, _TILE), 1)
    diff = (row - col).astype(f32)
    L = jnp.where(diff >= 0.0, _S * jnp.exp(log1ms * diff), 0.0)
    dvec = jnp.exp(
        log1ms
        * (jax.lax.broadcasted_iota(jnp.int32, (_TILE, 1), 0) + 1).astype(f32)
    )
    neg_droot = -(_DELTA ** _R)

    @pl.when(pl.program_id(1) == 0)
    def _init():
        carry_ref[...] = x_ref[0, 0:1, :]

    carry = carry_ref[...]
    for k in range(_TC // _TILE):
        v = x_ref[0, k * _TILE:(k + 1) * _TILE, :]
        p = jax.lax.dot(L, v, preferred_element_type=f32)
        m = p + dvec * carry
        carry = m[_TILE - 1:_TILE, :]
        o_ref[0, k * _TILE:(k + 1) * _TILE, :] = (
            jnp.sqrt(v * jnp.exp(-_ALPHA * jnp.log(m + _EPS)) + _DELTA)
            + neg_droot
        )
    carry_ref[...] = carry


@jax.jit
def kernel(x):
    B, T, F = x.shape
    grid = (B, T // _TC)
    return pl.pallas_call(
        _pcen_kernel,
        grid=grid,
        in_specs=[pl.BlockSpec((1, _TC, F), lambda b, t: (b, t, 0))],
        out_specs=pl.BlockSpec((1, _TC, F), lambda b, t: (b, t, 0)),
        out_shape=jax.ShapeDtypeStruct((B, T, F), x.dtype),
        scratch_shapes=[pltpu.VMEM((1, F), jnp.float32)],
        compiler_params=pltpu.CompilerParams(
            dimension_semantics=("parallel", "arbitrary"),
        ),
    )(x)
